# baseline (speedup 1.0000x reference)
import jax
import jax.numpy as jnp
from jax import lax
from jax.experimental import pallas as pl
from jax.experimental.pallas import tpu as pltpu

NUM_ROWS = 128
ROW_LEN = 100000
BLOCK_ROWS = 8
NUM_BLOCKS = NUM_ROWS // BLOCK_ROWS


def _body(x_ref, out_ref):
    t = x_ref[...]
    col = lax.broadcasted_iota(jnp.int32, (BLOCK_ROWS, ROW_LEN), 1)
    bmax = jnp.max(t, axis=1, keepdims=True)
    cand = jnp.where(t == bmax, col, ROW_LEN)
    out_ref[...] = jnp.min(cand, axis=1, keepdims=True)


@jax.jit
def _argmax_impl(logits):
    out = pl.pallas_call(
        _body,
        grid=(NUM_BLOCKS,),
        in_specs=[
            pl.BlockSpec((BLOCK_ROWS, ROW_LEN), lambda i: (i, 0)),
        ],
        out_specs=pl.BlockSpec((BLOCK_ROWS, 1), lambda i: (i, 0)),
        out_shape=jax.ShapeDtypeStruct((NUM_ROWS, 1), jnp.int32),
    )(logits)
    return out.reshape(NUM_ROWS)


def kernel(logits, temperatures):
    return _argmax_impl(logits)


# speedup vs baseline: 1.1343x; 1.1343x over previous
import jax
import jax.numpy as jnp
from jax import lax
from jax.experimental import pallas as pl
from jax.experimental.pallas import tpu as pltpu

NUM_ROWS = 128
ROW_LEN = 100000
NSTREAM = 4
BLOCK_COLS = 2048
STEP_COLS = NSTREAM * BLOCK_COLS
NUM_STEPS = ROW_LEN // STEP_COLS
TAIL_BLOCK = NUM_STEPS * NSTREAM


def _scan_block(t, colg, bv, bi):
    bmax = jnp.max(t, axis=1, keepdims=True)
    cand = jnp.where(t == bmax, colg, ROW_LEN)
    barg = jnp.min(cand, axis=1, keepdims=True)
    better = bmax > bv
    return jnp.where(better, bmax, bv), jnp.where(better, barg, bi)


def _body(x0_ref, x1_ref, x2_ref, x3_ref, xt_ref, out_ref, vmax_ref, vidx_ref):
    j = pl.program_id(0)

    @pl.when(j == 0)
    def _():
        vmax_ref[...] = jnp.full((NUM_ROWS, 1), -jnp.inf, jnp.float32)
        vidx_ref[...] = jnp.zeros((NUM_ROWS, 1), jnp.int32)

    col = lax.broadcasted_iota(jnp.int32, (NUM_ROWS, BLOCK_COLS), 1)
    bv = vmax_ref[...]
    bi = vidx_ref[...]
    for k, ref in enumerate((x0_ref, x1_ref, x2_ref, x3_ref)):
        colg = col + (j * NSTREAM + k) * BLOCK_COLS
        bv, bi = _scan_block(ref[...], colg, bv, bi)
    vmax_ref[...] = bv
    vidx_ref[...] = bi

    @pl.when(j == NUM_STEPS - 1)
    def _():
        colg = col + TAIL_BLOCK * BLOCK_COLS
        t = jnp.where(colg < ROW_LEN, xt_ref[...], -jnp.inf)
        fv, fi = _scan_block(t, colg, vmax_ref[...], vidx_ref[...])
        out_ref[...] = fi


@jax.jit
def _argmax_impl(logits):
    in_specs = [
        pl.BlockSpec(
            (NUM_ROWS, BLOCK_COLS),
            (lambda j, k=k: (0, j * NSTREAM + k)),
        )
        for k in range(NSTREAM)
    ] + [
        pl.BlockSpec((NUM_ROWS, BLOCK_COLS), lambda j: (0, TAIL_BLOCK)),
    ]
    out = pl.pallas_call(
        _body,
        grid=(NUM_STEPS,),
        in_specs=in_specs,
        out_specs=pl.BlockSpec((NUM_ROWS, 1), lambda j: (0, 0)),
        out_shape=jax.ShapeDtypeStruct((NUM_ROWS, 1), jnp.int32),
        scratch_shapes=[
            pltpu.VMEM((NUM_ROWS, 1), jnp.float32),
            pltpu.VMEM((NUM_ROWS, 1), jnp.int32),
        ],
    )(logits, logits, logits, logits, logits)
    return out.reshape(NUM_ROWS)


def kernel(logits, temperatures):
    return _argmax_impl(logits)
